# Initial kernel scaffold; baseline (speedup 1.0000x reference)
import functools

import jax
import jax.numpy as jnp
from jax import lax
from jax.experimental import pallas as pl
from jax.experimental.pallas import tpu as pltpu
from jax.experimental.pallas import tpu_sc as plsc

KS = 2
B_, H_, W_, C_ = 2, 256, 256, 96
N = B_ * H_ * W_ * C_
TOTAL = B_ * H_ * KS * W_ * KS * C_
SHIFT = 20
RNG = 1 << SHIFT
NB = TOTAL >> SHIFT
NC, NS, L = 2, 16, 16
NW = NC * NS
PER_TILE = N // NW
CHUNK = 2048
RPC = CHUNK // 128
NCHUNKS = PER_TILE // CHUNK
NROWS = N // 128
CNT = NB * NW * L
BPC = NB // NC
SEG = RNG // NS
ZB = 16384

_mesh = plsc.VectorSubcoreMesh(
    core_axis_name="c", subcore_axis_name="s", num_cores=NC, num_subcores=NS)


def _wid():
    return lax.axis_index("s") * NC + lax.axis_index("c")


def _lane0(v):
    lane = lax.iota(jnp.int32, L)
    return jnp.sum(jnp.where(lane == 0, v, 0))


@functools.partial(
    pl.kernel,
    out_type=jax.ShapeDtypeStruct((CNT,), jnp.int32),
    mesh=_mesh,
    scratch_types=[
        pltpu.VMEM((RPC, 128), jnp.int32),
        pltpu.VMEM((NB * L,), jnp.int32),
    ],
)
def _count_kernel(idx_hbm, cnt_hbm, win, hist):
    w = _wid()
    lane = lax.iota(jnp.int32, L)
    ones = jnp.ones((L,), jnp.int32)
    zeros = jnp.zeros((L,), jnp.int32)

    def _z(b, c):
        hist[pl.ds(b * L, L)] = zeros
        return c

    lax.fori_loop(0, NB, _z, 0)
    row0 = w * (PER_TILE // 128)

    def _chunk(ci, c):
        pltpu.sync_copy(idx_hbm.at[pl.ds(row0 + ci * RPC, RPC)], win)

        def _vec(j, cc):
            v = win[j // 8, pl.ds((j % 8) * L, L)]
            b = jnp.right_shift(v, SHIFT)
            plsc.addupdate_scatter(hist, [b * L + lane], ones)
            return cc

        lax.fori_loop(0, CHUNK // L, _vec, 0)
        return c

    lax.fori_loop(0, NCHUNKS, _chunk, 0)

    def _w(b, c):
        pltpu.sync_copy(hist.at[pl.ds(b * L, L)],
                        cnt_hbm.at[pl.ds((b * NW + w) * L, L)])
        return c

    lax.fori_loop(0, NB, _w, 0)


@functools.partial(
    pl.kernel,
    out_type=jax.ShapeDtypeStruct((CNT + L,), jnp.int32),
    mesh=_mesh,
    scratch_types=[
        pltpu.VMEM((CNT,), jnp.int32),
        pltpu.VMEM((CNT + L,), jnp.int32),
    ],
)
def _scan_kernel(cnt_hbm, offs_hbm, cbuf, obuf):
    w = _wid()

    @pl.when(w == 0)
    def _():
        pltpu.sync_copy(cnt_hbm, cbuf)

        def _step(i, carry):
            x = cbuf[pl.ds(i * L, L)]
            incl = plsc.cumsum(x)
            obuf[pl.ds(i * L, L)] = incl - x + carry
            return carry + jnp.sum(x)

        lax.fori_loop(0, CNT // L, _step, jnp.int32(0))
        obuf[pl.ds(CNT, L)] = jnp.full((L,), N, jnp.int32)
        pltpu.sync_copy(obuf, offs_hbm)


@functools.partial(
    pl.kernel,
    out_type=[
        jax.ShapeDtypeStruct((N,), jnp.int32),
        jax.ShapeDtypeStruct((N,), jnp.float32),
    ],
    mesh=_mesh,
    scratch_types=[
        pltpu.VMEM((RPC, 128), jnp.int32),
        pltpu.VMEM((RPC, 128), jnp.float32),
        pltpu.VMEM((RPC, 128), jnp.int32),
        pltpu.VMEM((NB * L,), jnp.int32),
    ],
)
def _permute_kernel(idx_hbm, val_hbm, offs_hbm, bidx_hbm, bval_hbm,
                    win_i, win_v, dest, own):
    w = _wid()
    lane = lax.iota(jnp.int32, L)

    def _lo(b, c):
        pltpu.sync_copy(offs_hbm.at[pl.ds((b * NW + w) * L, L)],
                        own.at[pl.ds(b * L, L)])
        return c

    lax.fori_loop(0, NB, _lo, 0)
    row0 = w * (PER_TILE // 128)

    def _chunk(ci, c):
        pltpu.sync_copy(idx_hbm.at[pl.ds(row0 + ci * RPC, RPC)], win_i)
        pltpu.sync_copy(val_hbm.at[pl.ds(row0 + ci * RPC, RPC)], win_v)

        def _vec(j, cc):
            r = j // 8
            col = (j % 8) * L
            v = win_i[r, pl.ds(col, L)]
            addr = jnp.right_shift(v, SHIFT) * L + lane
            cur = plsc.load_gather(own, [addr])
            plsc.store_scatter(own, [addr], cur + 1)
            dest[r, pl.ds(col, L)] = cur
            return cc

        lax.fori_loop(0, CHUNK // L, _vec, 0)
        pltpu.sync_copy(win_i, bidx_hbm.at[dest])
        pltpu.sync_copy(win_v, bval_hbm.at[dest])
        return c

    lax.fori_loop(0, NCHUNKS, _chunk, 0)


@functools.partial(
    pl.kernel,
    out_type=jax.ShapeDtypeStruct((TOTAL,), jnp.float32),
    mesh=_mesh,
    scratch_types=[
        pltpu.VMEM((RPC, 128), jnp.int32),
        pltpu.VMEM((RPC, 128), jnp.float32),
        pltpu.VMEM((RPC, 128), jnp.int32),
        pltpu.VMEM((RPC, 128), jnp.float32),
        pltpu.VMEM((ZB,), jnp.float32),
        pltpu.VMEM((CNT + L,), jnp.int32),
        pltpu.VMEM_SHARED((RNG,), jnp.float32),
    ],
)
def _accum_kernel(bidx_hbm, bval_hbm, offs_hbm, out_hbm,
                  win_i, win_v, sidx, sval, zbuf, offs_v, acc):
    core = lax.axis_index("c")
    sid = lax.axis_index("s")
    lane = lax.iota(jnp.int32, L)
    pltpu.sync_copy(offs_hbm, offs_v)
    fzeros = jnp.zeros((L,), jnp.float32)

    def _zz(i, c):
        zbuf[pl.ds(i * L, L)] = fzeros
        return c

    lax.fori_loop(0, ZB // L, _zz, 0)

    def _bucket(jb, c):
        b = core * BPC + jb
        lo = b * RNG
        start = _lane0(offs_v[pl.ds(b * NW * L, L)])
        end = _lane0(offs_v[pl.ds((b + 1) * NW * L, L)])

        def _zseg(q, cc):
            pltpu.sync_copy(zbuf, acc.at[pl.ds(sid * SEG + q * ZB, ZB)])
            return cc

        lax.fori_loop(0, SEG // ZB, _zseg, 0)
        plsc.subcore_barrier()

        c1 = (end + CHUNK - 1) // CHUNK

        def _cond(ci):
            return ci < c1

        def _body(ci):
            row = ci * RPC
            pltpu.sync_copy(bidx_hbm.at[pl.ds(row, RPC)], win_i)
            pltpu.sync_copy(bval_hbm.at[pl.ds(row, RPC)], win_v)

            def _vec(j, cc):
                r = j // 8
                col = (j % 8) * L
                lv = win_i[r, pl.ds(col, L)] - lo
                m = (lv >= 0) & (lv < RNG)
                sidx[r, pl.ds(col, L)] = jnp.where(m, lv, lane)
                sval[r, pl.ds(col, L)] = jnp.where(
                    m, win_v[r, pl.ds(col, L)], 0.0)
                return cc

            lax.fori_loop(0, CHUNK // L, _vec, 0)
            pltpu.sync_copy(sval, acc.at[sidx], add=True)
            return ci + NS

        lax.while_loop(_cond, _body, start // CHUNK + sid)
        plsc.subcore_barrier()

        def _fl(q, cc):
            pltpu.sync_copy(acc.at[pl.ds(sid * SEG + q * ZB, ZB)],
                            out_hbm.at[pl.ds(lo + sid * SEG + q * ZB, ZB)])
            return cc

        lax.fori_loop(0, SEG // ZB, _fl, 0)
        plsc.subcore_barrier()
        return c

    lax.fori_loop(0, BPC, _bucket, 0)


def kernel(input, ind):
    vals = input.reshape(-1)
    idx = ind.reshape(-1).astype(jnp.int32)
    idx2 = idx.reshape(NROWS, 128)
    val2 = vals.reshape(NROWS, 128)
    cnt = _count_kernel(idx2)
    offs = _scan_kernel(cnt)
    bidx, bval = _permute_kernel(idx2, val2, offs)
    out = _accum_kernel(bidx.reshape(NROWS, 128), bval.reshape(NROWS, 128),
                        offs)
    return out.reshape(B_, H_ * KS, W_ * KS, C_)


# speedup vs baseline: 1.4048x; 1.4048x over previous
import functools

import jax
import jax.numpy as jnp
from jax import lax
from jax.experimental import pallas as pl
from jax.experimental.pallas import tpu as pltpu
from jax.experimental.pallas import tpu_sc as plsc

KS = 2
B_, H_, W_, C_ = 2, 256, 256, 96
N = B_ * H_ * W_ * C_
TOTAL = B_ * H_ * KS * W_ * KS * C_
SHIFT = 20
RNG = 1 << SHIFT
NB = TOTAL >> SHIFT
NC, NS, L = 2, 16, 16
NW = NC * NS
PER_TILE = N // NW
CHUNK = 2048
RPC = CHUNK // 128
NCHUNKS = PER_TILE // CHUNK
NROWS = N // 128
CNT = NB * NW * L
BPC = NB // NC
SEG = RNG // NS
ZB = 16384

_mesh = plsc.VectorSubcoreMesh(
    core_axis_name="c", subcore_axis_name="s", num_cores=NC, num_subcores=NS)


def _wid():
    return lax.axis_index("s") * NC + lax.axis_index("c")


def _lane0(v):
    lane = lax.iota(jnp.int32, L)
    return jnp.sum(jnp.where(lane == 0, v, 0))


@functools.partial(
    pl.kernel,
    out_type=jax.ShapeDtypeStruct((CNT,), jnp.int32),
    mesh=_mesh,
    compiler_params=pltpu.CompilerParams(needs_layout_passes=False),
    scratch_types=[
        pltpu.VMEM((CHUNK,), jnp.int32),
        pltpu.VMEM((NB * L,), jnp.int32),
    ],
)
def _count_kernel(idx_hbm, cnt_hbm, win, hist):
    w = _wid()
    lane = lax.iota(jnp.int32, L)
    ones = jnp.ones((L,), jnp.int32)
    zeros = jnp.zeros((L,), jnp.int32)

    def _z(b, c):
        hist[pl.ds(b * L, L)] = zeros
        return c

    lax.fori_loop(0, NB, _z, 0)
    base = w * PER_TILE

    def _chunk(ci, c):
        pltpu.sync_copy(idx_hbm.at[pl.ds(base + ci * CHUNK, CHUNK)], win)

        def _vec(j, cc):
            v = win[pl.ds(j * L, L)]
            b = jnp.right_shift(v, SHIFT)
            plsc.addupdate_scatter(hist, [b * L + lane], ones)
            return cc

        lax.fori_loop(0, CHUNK // L, _vec, 0)
        return c

    lax.fori_loop(0, NCHUNKS, _chunk, 0)

    def _w(b, c):
        pltpu.sync_copy(hist.at[pl.ds(b * L, L)],
                        cnt_hbm.at[pl.ds((b * NW + w) * L, L)])
        return c

    lax.fori_loop(0, NB, _w, 0)


@functools.partial(
    pl.kernel,
    out_type=jax.ShapeDtypeStruct((CNT + L,), jnp.int32),
    mesh=_mesh,
    compiler_params=pltpu.CompilerParams(needs_layout_passes=False),
    scratch_types=[
        pltpu.VMEM((CNT,), jnp.int32),
        pltpu.VMEM((CNT + L,), jnp.int32),
    ],
)
def _scan_kernel(cnt_hbm, offs_hbm, cbuf, obuf):
    w = _wid()

    @pl.when(w == 0)
    def _():
        pltpu.sync_copy(cnt_hbm, cbuf)

        def _step(i, carry):
            x = cbuf[pl.ds(i * L, L)]
            incl = plsc.cumsum(x)
            obuf[pl.ds(i * L, L)] = incl - x + carry
            return carry + jnp.sum(x)

        lax.fori_loop(0, CNT // L, _step, jnp.int32(0))
        obuf[pl.ds(CNT, L)] = jnp.full((L,), N, jnp.int32)
        pltpu.sync_copy(obuf, offs_hbm)


@functools.partial(
    pl.kernel,
    out_type=[
        jax.ShapeDtypeStruct((N,), jnp.int32),
        jax.ShapeDtypeStruct((N,), jnp.float32),
    ],
    mesh=_mesh,
    compiler_params=pltpu.CompilerParams(needs_layout_passes=False),
    scratch_types=[
        pltpu.VMEM((CHUNK,), jnp.int32),
        pltpu.VMEM((CHUNK,), jnp.float32),
        pltpu.VMEM((CHUNK,), jnp.int32),
        pltpu.VMEM((NB * L,), jnp.int32),
    ],
)
def _permute_kernel(idx_hbm, val_hbm, offs_hbm, bidx_hbm, bval_hbm,
                    win_i, win_v, dest, own):
    w = _wid()
    lane = lax.iota(jnp.int32, L)

    def _lo(b, c):
        pltpu.sync_copy(offs_hbm.at[pl.ds((b * NW + w) * L, L)],
                        own.at[pl.ds(b * L, L)])
        return c

    lax.fori_loop(0, NB, _lo, 0)
    base = w * PER_TILE

    def _chunk(ci, c):
        pltpu.sync_copy(idx_hbm.at[pl.ds(base + ci * CHUNK, CHUNK)], win_i)
        pltpu.sync_copy(val_hbm.at[pl.ds(base + ci * CHUNK, CHUNK)], win_v)

        def _vec(j, cc):
            v = win_i[pl.ds(j * L, L)]
            addr = jnp.right_shift(v, SHIFT) * L + lane
            cur = plsc.load_gather(own, [addr])
            plsc.store_scatter(own, [addr], cur + 1)
            dest[pl.ds(j * L, L)] = cur
            return cc

        lax.fori_loop(0, CHUNK // L, _vec, 0)
        pltpu.sync_copy(win_i, bidx_hbm.at[dest])
        pltpu.sync_copy(win_v, bval_hbm.at[dest])
        return c

    lax.fori_loop(0, NCHUNKS, _chunk, 0)


@functools.partial(
    pl.kernel,
    out_type=jax.ShapeDtypeStruct((TOTAL,), jnp.float32),
    mesh=_mesh,
    compiler_params=pltpu.CompilerParams(needs_layout_passes=False),
    scratch_types=[
        pltpu.VMEM((CHUNK,), jnp.int32),
        pltpu.VMEM((CHUNK,), jnp.float32),
        pltpu.VMEM((CHUNK,), jnp.int32),
        pltpu.VMEM((CHUNK,), jnp.float32),
        pltpu.VMEM((ZB,), jnp.float32),
        pltpu.VMEM((CNT + L,), jnp.int32),
        pltpu.VMEM_SHARED((RNG,), jnp.float32),
    ],
)
def _accum_kernel(bidx_hbm, bval_hbm, offs_hbm, out_hbm,
                  win_i, win_v, sidx, sval, zbuf, offs_v, acc):
    core = lax.axis_index("c")
    sid = lax.axis_index("s")
    lane = lax.iota(jnp.int32, L)
    pltpu.sync_copy(offs_hbm, offs_v)
    fzeros = jnp.zeros((L,), jnp.float32)

    def _zz(i, c):
        zbuf[pl.ds(i * L, L)] = fzeros
        return c

    lax.fori_loop(0, ZB // L, _zz, 0)

    def _bucket(jb, c):
        b = core * BPC + jb
        lo = b * RNG
        start = _lane0(offs_v[pl.ds(b * NW * L, L)])
        end = _lane0(offs_v[pl.ds((b + 1) * NW * L, L)])

        def _zseg(q, cc):
            pltpu.sync_copy(zbuf, acc.at[pl.ds(sid * SEG + q * ZB, ZB)])
            return cc

        lax.fori_loop(0, SEG // ZB, _zseg, 0)
        plsc.subcore_barrier()

        c1 = (end + CHUNK - 1) // CHUNK

        def _cond(ci):
            return ci < c1

        def _body(ci):
            el = ci * CHUNK
            pltpu.sync_copy(bidx_hbm.at[pl.ds(el, CHUNK)], win_i)
            pltpu.sync_copy(bval_hbm.at[pl.ds(el, CHUNK)], win_v)

            def _vec(j, cc):
                lv = win_i[pl.ds(j * L, L)] - lo
                m = (lv >= 0) & (lv < RNG)
                sidx[pl.ds(j * L, L)] = jnp.where(m, lv, lane)
                sval[pl.ds(j * L, L)] = jnp.where(
                    m, win_v[pl.ds(j * L, L)], 0.0)
                return cc

            lax.fori_loop(0, CHUNK // L, _vec, 0)
            pltpu.sync_copy(sval, acc.at[sidx], add=True)
            return ci + NS

        lax.while_loop(_cond, _body, start // CHUNK + sid)
        plsc.subcore_barrier()

        def _fl(q, cc):
            pltpu.sync_copy(acc.at[pl.ds(sid * SEG + q * ZB, ZB)],
                            out_hbm.at[pl.ds(lo + sid * SEG + q * ZB, ZB)])
            return cc

        lax.fori_loop(0, SEG // ZB, _fl, 0)
        plsc.subcore_barrier()
        return c

    lax.fori_loop(0, BPC, _bucket, 0)


def kernel(input, ind):
    vals = input.reshape(-1)
    idx = ind.reshape(-1).astype(jnp.int32)
    cnt = _count_kernel(idx)
    offs = _scan_kernel(cnt)
    bidx, bval = _permute_kernel(idx, vals, offs)
    out = _accum_kernel(bidx, bval, offs)
    return out.reshape(B_, H_ * KS, W_ * KS, C_)


# speedup vs baseline: 11.5483x; 8.2206x over previous
import functools

import jax
import jax.numpy as jnp
from jax import lax
from jax.experimental import pallas as pl
from jax.experimental.pallas import tpu as pltpu
from jax.experimental.pallas import tpu_sc as plsc

KS = 2
B_, H_, W_, C_ = 2, 256, 256, 96
N = B_ * H_ * W_ * C_
TOTAL = B_ * H_ * KS * W_ * KS * C_
SHIFT = 20
RNG = 1 << SHIFT
NB = TOTAL >> SHIFT
NC, NS, L = 2, 16, 16
HALF = N // NC
NSW = 8
NG = NC * NSW
SWLEN = HALF // NSW
SUB = SWLEN // NS
CHUNK = 2048
NCH_SW = SUB // CHUNK
CNT = NG * NB * NS * L
GBLK = NB * NS * L
GRP_MAX = ((SWLEN + NB * 64 + 32767) // 32768) * 32768
N_PAD = NG * GRP_MAX
BPC = NB // NC
SEG = RNG // NS
ZB = 16384
ZSP = GRP_MAX // NS
ZB2 = ZSP // 4

_mesh = plsc.VectorSubcoreMesh(
    core_axis_name="c", subcore_axis_name="s", num_cores=NC, num_subcores=NS)


def _lane0(v):
    lane = lax.iota(jnp.int32, L)
    return jnp.sum(jnp.where(lane == 0, v, 0))


@functools.partial(
    pl.kernel,
    out_type=jax.ShapeDtypeStruct((CNT,), jnp.int32),
    mesh=_mesh,
    compiler_params=pltpu.CompilerParams(needs_layout_passes=False),
    scratch_types=[
        pltpu.VMEM((CHUNK,), jnp.int32),
        pltpu.VMEM((NB * L,), jnp.int32),
    ],
)
def _count_kernel(idx_hbm, cnt_hbm, win, hist):
    core = lax.axis_index("c")
    t = lax.axis_index("s")
    lane = lax.iota(jnp.int32, L)
    ones = jnp.ones((L,), jnp.int32)
    zeros = jnp.zeros((L,), jnp.int32)

    def _sw(sw, c0):
        g = core * NSW + sw
        base = core * HALF + sw * SWLEN + t * SUB

        def _z(b, c):
            hist[pl.ds(b * L, L)] = zeros
            return c

        lax.fori_loop(0, NB, _z, 0)

        def _chunk(ci, c):
            pltpu.sync_copy(idx_hbm.at[pl.ds(base + ci * CHUNK, CHUNK)], win)

            def _vec(j, cc):
                v = win[pl.ds(j * L, L)]
                b = jnp.right_shift(v, SHIFT)
                plsc.addupdate_scatter(hist, [b * L + lane], ones)
                return cc

            lax.fori_loop(0, CHUNK // L, _vec, 0)
            return c

        lax.fori_loop(0, NCH_SW, _chunk, 0)

        def _w(b, c):
            pltpu.sync_copy(
                hist.at[pl.ds(b * L, L)],
                cnt_hbm.at[pl.ds(((g * NB + b) * NS + t) * L, L)])
            return c

        lax.fori_loop(0, NB, _w, 0)
        return c0

    lax.fori_loop(0, NSW, _sw, 0)


@functools.partial(
    pl.kernel,
    out_type=jax.ShapeDtypeStruct((CNT + L,), jnp.int32),
    mesh=_mesh,
    compiler_params=pltpu.CompilerParams(needs_layout_passes=False),
    scratch_types=[
        pltpu.VMEM((GBLK,), jnp.int32),
        pltpu.VMEM((L,), jnp.int32),
    ],
)
def _scan_kernel(cnt_hbm, offs_hbm, gbuf, tail):
    core = lax.axis_index("c")
    t = lax.axis_index("s")

    @pl.when((core == 0) & (t == 0))
    def _():
        def _grp(g, carry):
            pltpu.sync_copy(cnt_hbm.at[pl.ds(g * GBLK, GBLK)], gbuf)

            def _bkt(b, carry_b):
                def _step(k, carry_k):
                    i = b * L + k
                    x = gbuf[pl.ds(i * L, L)]
                    incl = plsc.cumsum(x)
                    gbuf[pl.ds(i * L, L)] = incl - x + carry_k
                    return carry_k + jnp.sum(x)

                ce = lax.fori_loop(0, NS, _step, carry_b)
                return jnp.bitwise_and(ce + 63, -64)

            ce = lax.fori_loop(0, NB, _bkt, carry)
            pltpu.sync_copy(gbuf, offs_hbm.at[pl.ds(g * GBLK, GBLK)])
            return jnp.bitwise_and(ce + 32767, -32768)

        total = lax.fori_loop(0, NG, _grp, jnp.int32(0))
        tail[pl.ds(0, L)] = jnp.full((L,), 1, jnp.int32) * total
        pltpu.sync_copy(tail, offs_hbm.at[pl.ds(CNT, L)])


@functools.partial(
    pl.kernel,
    out_type=[
        jax.ShapeDtypeStruct((N_PAD,), jnp.int32),
        jax.ShapeDtypeStruct((N_PAD,), jnp.float32),
    ],
    mesh=_mesh,
    compiler_params=pltpu.CompilerParams(needs_layout_passes=False),
    scratch_types=[
        pltpu.VMEM((CHUNK,), jnp.int32),
        pltpu.VMEM((CHUNK,), jnp.float32),
        pltpu.VMEM((CHUNK,), jnp.int32),
        pltpu.VMEM((NB * L,), jnp.int32),
        pltpu.VMEM((L,), jnp.int32),
        pltpu.VMEM((ZB2,), jnp.float32),
        pltpu.VMEM_SHARED((GRP_MAX,), jnp.int32),
        pltpu.VMEM_SHARED((GRP_MAX,), jnp.float32),
    ],
)
def _bin_kernel(idx_hbm, val_hbm, offs_hbm, bidx_hbm, bval_hbm,
                win_i, win_v, dest, own, g16, zbuf, sp_i, sp_v):
    core = lax.axis_index("c")
    t = lax.axis_index("s")
    lane = lax.iota(jnp.int32, L)
    fzeros = jnp.zeros((L,), jnp.float32)

    def _zz(i, c):
        zbuf[pl.ds(i * L, L)] = fzeros
        return c

    lax.fori_loop(0, ZB2 // L, _zz, 0)

    def _sw(sw, c0):
        g = core * NSW + sw
        base = core * HALF + sw * SWLEN + t * SUB
        pltpu.sync_copy(offs_hbm.at[pl.ds(g * GBLK, L)], g16)
        gb = pl.multiple_of(_lane0(g16[pl.ds(0, L)]), 2048)
        pltpu.sync_copy(offs_hbm.at[pl.ds((g + 1) * GBLK, L)], g16)
        gn = pl.multiple_of(_lane0(g16[pl.ds(0, L)]), 2048)

        def _zs(q, c):
            pltpu.sync_copy(zbuf, sp_v.at[pl.ds(t * ZSP + q * ZB2, ZB2)])
            return c

        lax.fori_loop(0, 4, _zs, 0)

        def _lo(b, c):
            pltpu.sync_copy(
                offs_hbm.at[pl.ds(((g * NB + b) * NS + t) * L, L)],
                own.at[pl.ds(b * L, L)])
            return c

        lax.fori_loop(0, NB, _lo, 0)
        plsc.subcore_barrier()

        def _chunk(ci, c):
            pltpu.sync_copy(idx_hbm.at[pl.ds(base + ci * CHUNK, CHUNK)],
                            win_i)
            pltpu.sync_copy(val_hbm.at[pl.ds(base + ci * CHUNK, CHUNK)],
                            win_v)

            def _vec(j, cc):
                v = win_i[pl.ds(j * L, L)]
                addr = jnp.right_shift(v, SHIFT) * L + lane
                cur = plsc.load_gather(own, [addr])
                plsc.store_scatter(own, [addr], cur + 1)
                dest[pl.ds(j * L, L)] = cur - gb
                return cc

            lax.fori_loop(0, CHUNK // L, _vec, 0)
            pltpu.sync_copy(win_i, sp_i.at[dest])
            pltpu.sync_copy(win_v, sp_v.at[dest])
            return c

        lax.fori_loop(0, NCH_SW, _chunk, 0)
        plsc.subcore_barrier()

        nch = (gn - gb) // CHUNK

        def _fcond(ch):
            return ch < nch

        def _fbody(ch):
            pltpu.sync_copy(sp_i.at[pl.ds(ch * CHUNK, CHUNK)],
                            bidx_hbm.at[pl.ds(gb + ch * CHUNK, CHUNK)])
            pltpu.sync_copy(sp_v.at[pl.ds(ch * CHUNK, CHUNK)],
                            bval_hbm.at[pl.ds(gb + ch * CHUNK, CHUNK)])
            return ch + NS

        lax.while_loop(_fcond, _fbody, t)
        plsc.subcore_barrier()
        return c0

    lax.fori_loop(0, NSW, _sw, 0)


@functools.partial(
    pl.kernel,
    out_type=jax.ShapeDtypeStruct((TOTAL,), jnp.float32),
    mesh=_mesh,
    compiler_params=pltpu.CompilerParams(needs_layout_passes=False),
    scratch_types=[
        pltpu.VMEM((CHUNK,), jnp.int32),
        pltpu.VMEM((CHUNK,), jnp.float32),
        pltpu.VMEM((CHUNK,), jnp.int32),
        pltpu.VMEM((CHUNK,), jnp.float32),
        pltpu.VMEM((ZB,), jnp.float32),
        pltpu.VMEM(((NB + 1) * L,), jnp.int32),
        pltpu.VMEM_SHARED((RNG,), jnp.float32),
    ],
)
def _accum_kernel(bidx_hbm, bval_hbm, offs_hbm, out_hbm,
                  win_i, win_v, sidx, sval, zbuf, bnd, acc):
    core = lax.axis_index("c")
    sid = lax.axis_index("s")
    lane = lax.iota(jnp.int32, L)
    fzeros = jnp.zeros((L,), jnp.float32)

    def _zz(i, c):
        zbuf[pl.ds(i * L, L)] = fzeros
        return c

    lax.fori_loop(0, ZB // L, _zz, 0)

    def _bn(b, c):
        pltpu.sync_copy(offs_hbm.at[pl.ds((sid * NB + b) * NS * L, L)],
                        bnd.at[pl.ds(b * L, L)])
        return c

    lax.fori_loop(0, NB + 1, _bn, 0)

    def _bucket(jb, c):
        b = core * BPC + jb
        lo = b * RNG

        def _zseg(q, cc):
            pltpu.sync_copy(zbuf, acc.at[pl.ds(sid * SEG + q * ZB, ZB)])
            return cc

        lax.fori_loop(0, SEG // ZB, _zseg, 0)
        plsc.subcore_barrier()

        ss = _lane0(bnd[pl.ds(b * L, L)])
        re = _lane0(bnd[pl.ds((b + 1) * L, L)])
        c1 = (re + CHUNK - 1) // CHUNK

        def _cond(ci):
            return ci < c1

        def _body(ci):
            el = ci * CHUNK
            pltpu.sync_copy(bidx_hbm.at[pl.ds(el, CHUNK)], win_i)
            pltpu.sync_copy(bval_hbm.at[pl.ds(el, CHUNK)], win_v)

            def _vec(j, cc):
                lv = win_i[pl.ds(j * L, L)] - lo
                m = (lv >= 0) & (lv < RNG)
                sidx[pl.ds(j * L, L)] = jnp.where(m, lv, lane)
                sval[pl.ds(j * L, L)] = jnp.where(
                    m, win_v[pl.ds(j * L, L)], 0.0)
                return cc

            lax.fori_loop(0, CHUNK // L, _vec, 0)
            pltpu.sync_copy(sval, acc.at[sidx], add=True)
            return ci + 1

        lax.while_loop(_cond, _body, ss // CHUNK)
        plsc.subcore_barrier()

        def _fl(q, cc):
            pltpu.sync_copy(acc.at[pl.ds(sid * SEG + q * ZB, ZB)],
                            out_hbm.at[pl.ds(lo + sid * SEG + q * ZB, ZB)])
            return cc

        lax.fori_loop(0, SEG // ZB, _fl, 0)
        plsc.subcore_barrier()
        return c

    lax.fori_loop(0, BPC, _bucket, 0)


def kernel(input, ind):
    vals = input.reshape(-1)
    idx = ind.reshape(-1).astype(jnp.int32)
    cnt = _count_kernel(idx)
    offs = _scan_kernel(cnt)
    bidx, bval = _bin_kernel(idx, vals, offs)
    out = _accum_kernel(bidx, bval, offs)
    return out.reshape(B_, H_ * KS, W_ * KS, C_)


# speedup vs baseline: 12.9889x; 1.1247x over previous
import functools

import jax
import jax.numpy as jnp
from jax import lax
from jax.experimental import pallas as pl
from jax.experimental.pallas import tpu as pltpu
from jax.experimental.pallas import tpu_sc as plsc

KS = 2
B_, H_, W_, C_ = 2, 256, 256, 96
N = B_ * H_ * W_ * C_
TOTAL = B_ * H_ * KS * W_ * KS * C_
SHIFT = 19
RNG = 1 << SHIFT
NB = TOTAL >> SHIFT
NC, NS, L = 2, 16, 16
HALF = N // NC
NSW = 16
NG = NC * NSW
SWLEN = HALF // NSW
SUB = SWLEN // NS
CHUNK = 2048
NCH_SW = SUB // CHUNK
CNT = NG * NB * NS * L
GBLK = NB * NS * L
GRP_MAX = ((SWLEN + NB * 64 + 32767) // 32768) * 32768
N_PAD = NG * GRP_MAX
BPC = NB // NC
SEG = RNG // NS
ZB = 16384
ZSP = GRP_MAX // NS
CH_B = 4096
NCHB = SUB // CH_B
FCH = 32768

_mesh = plsc.VectorSubcoreMesh(
    core_axis_name="c", subcore_axis_name="s", num_cores=NC, num_subcores=NS)


def _lane0(v):
    lane = lax.iota(jnp.int32, L)
    return jnp.sum(jnp.where(lane == 0, v, 0))


@functools.partial(
    pl.kernel,
    out_type=jax.ShapeDtypeStruct((CNT,), jnp.int32),
    mesh=_mesh,
    compiler_params=pltpu.CompilerParams(needs_layout_passes=False),
    scratch_types=[
        pltpu.VMEM((CH_B,), jnp.int32),
        pltpu.VMEM((CH_B,), jnp.int32),
        pltpu.VMEM((NB * L,), jnp.int32),
        pltpu.SemaphoreType.DMA,
        pltpu.SemaphoreType.DMA,
    ],
)
def _count_kernel(idx_hbm, cnt_hbm, win0, win1, hist, sem0, sem1):
    core = lax.axis_index("c")
    t = lax.axis_index("s")
    lane = lax.iota(jnp.int32, L)
    ones = jnp.ones((L,), jnp.int32)
    zeros = jnp.zeros((L,), jnp.int32)
    wins = (win0, win1)
    sems = (sem0, sem1)

    def _sw(sw, c0):
        g = core * NSW + sw
        base = core * HALF + sw * SWLEN + t * SUB

        def _z(b, c):
            hist[pl.ds(b * L, L)] = zeros
            return c

        lax.fori_loop(0, NB, _z, 0)

        ld = [None] * NCHB
        ld[0] = pltpu.async_copy(
            idx_hbm.at[pl.ds(base, CH_B)], wins[0], sems[0])
        for ci in range(NCHB):
            b = ci % 2
            ld[ci].wait()
            if ci + 1 < NCHB:
                nb = (ci + 1) % 2
                ld[ci + 1] = pltpu.async_copy(
                    idx_hbm.at[pl.ds(base + (ci + 1) * CH_B, CH_B)],
                    wins[nb], sems[nb])

            def _vec(j, cc):
                v = wins[b][pl.ds(j * L, L)]
                bk = jnp.right_shift(v, SHIFT)
                plsc.addupdate_scatter(hist, [bk * L + lane], ones)
                return cc

            lax.fori_loop(0, CH_B // L, _vec, 0)

        wd = []
        for b in range(NB):
            wd.append(pltpu.async_copy(
                hist.at[pl.ds(b * L, L)],
                cnt_hbm.at[pl.ds(((g * NB + b) * NS + t) * L, L)], sem0))
        for d in wd:
            d.wait()
        return c0

    lax.fori_loop(0, NSW, _sw, 0)


@functools.partial(
    pl.kernel,
    out_type=jax.ShapeDtypeStruct((CNT + L,), jnp.int32),
    mesh=_mesh,
    compiler_params=pltpu.CompilerParams(needs_layout_passes=False),
    scratch_types=[
        pltpu.VMEM((2 * GBLK,), jnp.int32),
        pltpu.VMEM((NS * L,), jnp.int32),
        pltpu.VMEM((L,), jnp.int32),
        pltpu.VMEM_SHARED((NS * L,), jnp.int32),
    ],
)
def _scan_kernel(cnt_hbm, offs_hbm, gbuf, gts, tail, sgt):
    core = lax.axis_index("c")
    t = lax.axis_index("s")

    @pl.when(core == 0)
    def _():
        pltpu.sync_copy(cnt_hbm.at[pl.ds(2 * t * GBLK, 2 * GBLK)], gbuf)

        def _grp(q, carry):
            def _bkt(b, carry_b):
                def _step(k, carry_k):
                    i = q * GBLK + b * NS * L + k * L
                    x = gbuf[pl.ds(i, L)]
                    incl = plsc.cumsum(x)
                    gbuf[pl.ds(i, L)] = incl - x + carry_k
                    return carry_k + jnp.sum(x)

                ce = lax.fori_loop(0, NS, _step, carry_b)
                return jnp.bitwise_and(ce + 63, -64)

            ce = lax.fori_loop(0, NB, _bkt, carry)
            return jnp.bitwise_and(ce + 32767, -32768)

        pair_total = lax.fori_loop(0, 2, _grp, jnp.int32(0))
        tail[pl.ds(0, L)] = jnp.full((L,), 1, jnp.int32) * pair_total
        pltpu.sync_copy(tail, sgt.at[pl.ds(t * L, L)])
        plsc.subcore_barrier()
        pltpu.sync_copy(sgt, gts)

        def _base(tp, acc2):
            v = _lane0(gts[pl.ds(tp * L, L)])
            return acc2 + jnp.where(tp < t, v, 0)

        base = lax.fori_loop(0, NS, _base, jnp.int32(0))

        def _add(i, c):
            gbuf[pl.ds(i * L, L)] = gbuf[pl.ds(i * L, L)] + base
            return c

        lax.fori_loop(0, 2 * GBLK // L, _add, 0)
        pltpu.sync_copy(gbuf, offs_hbm.at[pl.ds(2 * t * GBLK, 2 * GBLK)])

        @pl.when(t == NS - 1)
        def _tail():
            tail[pl.ds(0, L)] = jnp.full((L,), 1, jnp.int32) * (
                base + pair_total)
            pltpu.sync_copy(tail, offs_hbm.at[pl.ds(CNT, L)])


@functools.partial(
    pl.kernel,
    out_type=[
        jax.ShapeDtypeStruct((N_PAD,), jnp.int32),
        jax.ShapeDtypeStruct((N_PAD,), jnp.float32),
    ],
    mesh=_mesh,
    compiler_params=pltpu.CompilerParams(needs_layout_passes=False),
    scratch_types=[
        pltpu.VMEM((CH_B,), jnp.int32),
        pltpu.VMEM((CH_B,), jnp.int32),
        pltpu.VMEM((CH_B,), jnp.float32),
        pltpu.VMEM((CH_B,), jnp.float32),
        pltpu.VMEM((CH_B,), jnp.int32),
        pltpu.VMEM((CH_B,), jnp.int32),
        pltpu.VMEM((NB * L,), jnp.int32),
        pltpu.VMEM((L,), jnp.int32),
        pltpu.VMEM((ZSP,), jnp.float32),
        pltpu.VMEM_SHARED((GRP_MAX,), jnp.int32),
        pltpu.VMEM_SHARED((GRP_MAX,), jnp.float32),
        pltpu.SemaphoreType.DMA,
        pltpu.SemaphoreType.DMA,
        pltpu.SemaphoreType.DMA,
        pltpu.SemaphoreType.DMA,
        pltpu.SemaphoreType.DMA,
        pltpu.SemaphoreType.DMA,
    ],
)
def _bin_kernel(idx_hbm, val_hbm, offs_hbm, bidx_hbm, bval_hbm,
                wi0, wi1, wv0, wv1, de0, de1, own, g16, zbuf, sp_i, sp_v,
                sli0, sli1, slv0, slv1, ssc0, ssc1):
    core = lax.axis_index("c")
    t = lax.axis_index("s")
    lane = lax.iota(jnp.int32, L)
    fzeros = jnp.zeros((L,), jnp.float32)
    wis = (wi0, wi1)
    wvs = (wv0, wv1)
    des = (de0, de1)
    slis = (sli0, sli1)
    slvs = (slv0, slv1)
    sscs = (ssc0, ssc1)

    def _zz(i, c):
        zbuf[pl.ds(i * L, L)] = fzeros
        return c

    lax.fori_loop(0, ZSP // L, _zz, 0)

    def _sw(sw, c0):
        g = core * NSW + sw
        base = core * HALF + sw * SWLEN + t * SUB
        pltpu.sync_copy(offs_hbm.at[pl.ds(g * GBLK, L)], g16)
        gb = pl.multiple_of(_lane0(g16[pl.ds(0, L)]), 2048)
        pltpu.sync_copy(offs_hbm.at[pl.ds((g + 1) * GBLK, L)], g16)
        gn = pl.multiple_of(_lane0(g16[pl.ds(0, L)]), 2048)

        zd = pltpu.async_copy(zbuf, sp_v.at[pl.ds(t * ZSP, ZSP)], ssc0)

        cd = []
        for b in range(NB):
            cd.append(pltpu.async_copy(
                offs_hbm.at[pl.ds(((g * NB + b) * NS + t) * L, L)],
                own.at[pl.ds(b * L, L)], sli0))
        for d in cd:
            d.wait()
        zd.wait()
        plsc.subcore_barrier()

        ld_i = [None] * NCHB
        ld_v = [None] * NCHB
        sc_i = [None] * NCHB
        sc_v = [None] * NCHB
        ld_i[0] = pltpu.async_copy(
            idx_hbm.at[pl.ds(base, CH_B)], wis[0], slis[0])
        ld_v[0] = pltpu.async_copy(
            val_hbm.at[pl.ds(base, CH_B)], wvs[0], slvs[0])
        for ci in range(NCHB):
            b = ci % 2
            ld_i[ci].wait()
            ld_v[ci].wait()
            if ci + 1 < NCHB:
                nb = (ci + 1) % 2
                if ci >= 1:
                    sc_i[ci - 1].wait()
                    sc_v[ci - 1].wait()
                ld_i[ci + 1] = pltpu.async_copy(
                    idx_hbm.at[pl.ds(base + (ci + 1) * CH_B, CH_B)],
                    wis[nb], slis[nb])
                ld_v[ci + 1] = pltpu.async_copy(
                    val_hbm.at[pl.ds(base + (ci + 1) * CH_B, CH_B)],
                    wvs[nb], slvs[nb])

            def _vec(j, cc):
                v = wis[b][pl.ds(j * L, L)]
                addr = jnp.right_shift(v, SHIFT) * L + lane
                cur = plsc.load_gather(own, [addr])
                plsc.store_scatter(own, [addr], cur + 1)
                des[b][pl.ds(j * L, L)] = cur - gb
                return cc

            lax.fori_loop(0, CH_B // L, _vec, 0)
            sc_i[ci] = pltpu.async_copy(wis[b], sp_i.at[des[b]], sscs[b])
            sc_v[ci] = pltpu.async_copy(wvs[b], sp_v.at[des[b]], sscs[b])
        for ci in (NCHB - 2, NCHB - 1):
            sc_i[ci].wait()
            sc_v[ci].wait()
        plsc.subcore_barrier()

        nch = (gn - gb) // FCH

        def _fcond(ch):
            return ch < nch

        def _fbody(ch):
            pltpu.sync_copy(sp_i.at[pl.ds(ch * FCH, FCH)],
                            bidx_hbm.at[pl.ds(gb + ch * FCH, FCH)])
            pltpu.sync_copy(sp_v.at[pl.ds(ch * FCH, FCH)],
                            bval_hbm.at[pl.ds(gb + ch * FCH, FCH)])
            return ch + NS

        lax.while_loop(_fcond, _fbody, t)
        plsc.subcore_barrier()
        return c0

    lax.fori_loop(0, NSW, _sw, 0)


@functools.partial(
    pl.kernel,
    out_type=jax.ShapeDtypeStruct((TOTAL,), jnp.float32),
    mesh=_mesh,
    compiler_params=pltpu.CompilerParams(needs_layout_passes=False),
    scratch_types=[
        pltpu.VMEM((CHUNK,), jnp.int32),
        pltpu.VMEM((CHUNK,), jnp.float32),
        pltpu.VMEM((CHUNK,), jnp.int32),
        pltpu.VMEM((CHUNK,), jnp.float32),
        pltpu.VMEM((SEG,), jnp.float32),
        pltpu.VMEM((2 * (NB + 1) * L,), jnp.int32),
        pltpu.VMEM_SHARED((RNG,), jnp.float32),
    ],
)
def _accum_kernel(bidx_hbm, bval_hbm, offs_hbm, out_hbm,
                  win_i, win_v, sidx, sval, zbuf, bnd, acc):
    core = lax.axis_index("c")
    sid = lax.axis_index("s")
    lane = lax.iota(jnp.int32, L)
    fzeros = jnp.zeros((L,), jnp.float32)

    def _zz(i, c):
        zbuf[pl.ds(i * L, L)] = fzeros
        return c

    lax.fori_loop(0, SEG // L, _zz, 0)

    def _bn(b, c):
        pltpu.sync_copy(offs_hbm.at[pl.ds((sid * NB + b) * NS * L, L)],
                        bnd.at[pl.ds(b * L, L)])
        pltpu.sync_copy(
            offs_hbm.at[pl.ds(((sid + NS) * NB + b) * NS * L, L)],
            bnd.at[pl.ds((NB + 1 + b) * L, L)])
        return c

    lax.fori_loop(0, NB + 1, _bn, 0)

    def _bucket(jb, c):
        b = core * BPC + jb
        lo = b * RNG

        pltpu.sync_copy(zbuf, acc.at[pl.ds(sid * SEG, SEG)])
        plsc.subcore_barrier()

        for q in range(2):
            ss = _lane0(bnd[pl.ds((q * (NB + 1) + b) * L, L)])
            re = _lane0(bnd[pl.ds((q * (NB + 1) + b + 1) * L, L)])
            c1 = (re + CHUNK - 1) // CHUNK

            def _body(ci):
                el = ci * CHUNK
                pltpu.sync_copy(bidx_hbm.at[pl.ds(el, CHUNK)], win_i)
                pltpu.sync_copy(bval_hbm.at[pl.ds(el, CHUNK)], win_v)

                def _vec(j, cc):
                    lv = win_i[pl.ds(j * L, L)] - lo
                    m = (lv >= 0) & (lv < RNG)
                    sidx[pl.ds(j * L, L)] = jnp.where(m, lv, lane)
                    sval[pl.ds(j * L, L)] = jnp.where(
                        m, win_v[pl.ds(j * L, L)], 0.0)
                    return cc

                lax.fori_loop(0, CHUNK // L, _vec, 0)
                pltpu.sync_copy(sval, acc.at[sidx], add=True)
                return ci + 1

            lax.while_loop(lambda ci: ci < c1, _body, ss // CHUNK)
        plsc.subcore_barrier()

        pltpu.sync_copy(acc.at[pl.ds(sid * SEG, SEG)],
                        out_hbm.at[pl.ds(lo + sid * SEG, SEG)])
        plsc.subcore_barrier()
        return c

    lax.fori_loop(0, BPC, _bucket, 0)


def kernel(input, ind):
    vals = input.reshape(-1)
    idx = ind.reshape(-1).astype(jnp.int32)
    cnt = _count_kernel(idx)
    offs = _scan_kernel(cnt)
    bidx, bval = _bin_kernel(idx, vals, offs)
    out = _accum_kernel(bidx, bval, offs)
    return out.reshape(B_, H_ * KS, W_ * KS, C_)
